# baseline (speedup 1.0000x reference)
import functools
import math

import jax
import jax.numpy as jnp
from jax import lax
from jax.experimental import pallas as pl
from jax.experimental.pallas import tpu as pltpu
from jax.experimental.pallas import tpu_sc as plsc

_info = plsc.get_sparse_core_info()
_NC = _info.num_cores
_NS = _info.num_subcores
_L = _info.num_lanes
_NW = _NC * _NS


@functools.lru_cache(maxsize=None)
def _make_emb(N, V, D):
    n_per_w = N // _NW
    C = 128
    n_chunks = n_per_w // C
    n_pairs = n_chunks // 2
    assert n_per_w * _NW == N and C * n_chunks == n_per_w and 2 * n_pairs == n_chunks
    scale = float(math.sqrt(D))
    mesh = plsc.VectorSubcoreMesh(core_axis_name="c", subcore_axis_name="s")

    @functools.partial(
        pl.kernel,
        out_type=jax.ShapeDtypeStruct((N, D), jnp.float32),
        mesh=mesh,
        compiler_params=pltpu.CompilerParams(use_tc_tiling_on_sc=False),
        scratch_types=[
            pltpu.VMEM((n_per_w,), jnp.int32),
            pltpu.VMEM((C, D), jnp.float32),
            pltpu.VMEM((C, D), jnp.float32),
            pltpu.SemaphoreType.DMA,
            pltpu.SemaphoreType.DMA,
        ],
    )
    def emb(tokens_hbm, table_hbm, out_hbm, idx_v, buf0, buf1, sem0, sem1):
        wid = lax.axis_index("s") * _NC + lax.axis_index("c")
        base = wid * n_per_w
        pltpu.sync_copy(tokens_hbm.at[pl.ds(base, n_per_w)], idx_v)

        def start(i, buf, sem):
            pltpu.async_copy(table_hbm.at[idx_v.at[pl.ds(i * C, C)]], buf, sem)

        def finish(i, buf, sem):
            pltpu.make_async_copy(
                table_hbm.at[idx_v.at[pl.ds(i * C, C)]], buf, sem
            ).wait()

            def scale_row(r, carry):
                for k in range(D // _L):
                    sl = pl.ds(k * _L, _L)
                    buf[r, sl] = buf[r, sl] * scale
                return carry

            lax.fori_loop(0, C, scale_row, 0, unroll=4)
            pltpu.sync_copy(buf, out_hbm.at[pl.ds(base + i * C, C)])

        start(0, buf0, sem0)

        def body(p, carry):
            i0 = 2 * p
            i1 = i0 + 1
            start(i1, buf1, sem1)
            finish(i0, buf0, sem0)

            @pl.when(p + 1 < n_pairs)
            def _():
                start(i1 + 1, buf0, sem0)

            finish(i1, buf1, sem1)
            return carry

        lax.fori_loop(0, n_pairs, body, 0)

    return emb


def kernel(tokens, table):
    B, T = tokens.shape
    V, D = table.shape
    N = B * T
    flat = tokens.reshape(N).astype(jnp.int32)
    out = _make_emb(N, V, D)(flat, table)
    return out.reshape(B, T, D)
